# baseline (speedup 1.0000x reference)
import functools

import jax
import jax.numpy as jnp
import numpy as np
from jax import lax
from jax.experimental import pallas as pl
from jax.experimental.pallas import tpu as pltpu
from jax.experimental.pallas import tpu_sc as plsc

VOCAB = 1000000
DIM = 64
B, S = 1024, 200
N_TOK = B * S
N_ROW = N_TOK * DIM // 1024
KEEP = np.float32(0.7)
THRESH = np.uint32(5872026 * 512)

SAMPLE_KEYS = np.array([
    (0x6D3E048F, 0x1022172D), (0x03D7B32D, 0xADD083F4),
    (0x92FB20EA, 0x0F38D913), (0xBAD56946, 0x354BA891),
    (0xB013AEE3, 0xC34EDDF6), (0xA4D91A96, 0x3122544E),
    (0xA506C508, 0xB6207291), (0x97D0552F, 0x51BF719F),
    (0x3C999167, 0x8E776FEA), (0x4349448B, 0x92D8BF3B),
], dtype=np.uint32)

_NC, _NS = 2, 16
_NW = _NC * _NS
_RPW = N_TOK // _NW
_CH = 128
_NCHUNK = _RPW // _CH


def _sc_gather(ids2d, table):
    mesh = plsc.VectorSubcoreMesh(core_axis_name="c", subcore_axis_name="s")

    @functools.partial(
        pl.kernel,
        mesh=mesh,
        out_type=jax.ShapeDtypeStruct((N_TOK, DIM), jnp.float32),
        scratch_types=[
            pltpu.VMEM((_NCHUNK, _CH), jnp.int32),
            pltpu.VMEM((2, _CH, DIM), jnp.float32),
            pltpu.SemaphoreType.DMA,
            pltpu.SemaphoreType.DMA,
        ],
        compiler_params=pltpu.CompilerParams(use_tc_tiling_on_sc=False),
    )
    def sc_kernel(ids_hbm, table_hbm, out_hbm, idx_v, rows_v, sem0, sem1):
        w = lax.axis_index("s") * _NC + lax.axis_index("c")
        pltpu.sync_copy(ids_hbm.at[w], idx_v)
        out_base = w * _RPW

        def start(k, buf, sem):
            pltpu.async_copy(table_hbm.at[idx_v.at[k]], rows_v.at[buf], sem)

        def wait(buf, sem):
            pltpu.make_async_copy(
                table_hbm.at[idx_v.at[0]], rows_v.at[buf], sem).wait()

        def drain(k, buf, sem):
            wait(buf, sem)
            off = pl.multiple_of(out_base + k * _CH, _CH)
            pltpu.sync_copy(rows_v.at[buf], out_hbm.at[pl.ds(off, _CH)])

        start(0, 0, sem0)
        start(1, 1, sem1)

        def body(t, _):
            k = 2 * t
            drain(k, 0, sem0)

            @pl.when(k + 2 < _NCHUNK)
            def _():
                start(k + 2, 0, sem0)

            drain(k + 1, 1, sem1)

            @pl.when(k + 3 < _NCHUNK)
            def _():
                start(k + 3, 1, sem1)
            return 0

        lax.fori_loop(0, _NCHUNK // 2, body, 0)

    return sc_kernel(ids2d, table)


_RB = 128
_NBLK = N_ROW // _RB
MC = 10


def _threefry_bits(j, k0, k1):
    ks2 = k0 ^ k1 ^ np.uint32(0x1BD11BDA)
    ks = (k0, k1, ks2)
    rot = ((13, 15, 26, 6), (17, 29, 16, 24))
    x0 = jnp.zeros_like(j) + k0
    x1 = j + k1
    for i in range(5):
        for r in rot[i % 2]:
            x0 = x0 + x1
            x1 = ((x1 << r) | (x1 >> (32 - r))) ^ x0
        x0 = x0 + ks[(i + 1) % 3]
        x1 = x1 + (ks[(i + 2) % 3] + np.uint32(i + 1))
    return x0 ^ x1


_SB = 8
_INV_KEEP = np.float32(1.0 / 0.7)


def _fused_body(keys_ref, emb_ref, samp_hbm, mean_ref, std_ref, tu_ref,
                sbuf, acc_ref, sem0, sem1):
    r = pl.program_id(0)
    base = pl.multiple_of(r * _RB, _RB)
    sems = (sem0, sem1)

    for m in range(MC):
        k0 = keys_ref[m, 0, 0]
        k1 = keys_ref[m, 0, 1]
        buf = m % 2
        if m >= 2:
            pltpu.make_async_copy(
                sbuf.at[buf], samp_hbm.at[m - 2, pl.ds(base, _RB), :],
                sems[buf]).wait()
        for s in range(_RB // _SB):
            sl = pl.ds(s * _SB, _SB)
            row = lax.broadcasted_iota(jnp.int32, (_SB, 1024), 0)
            col = lax.broadcasted_iota(jnp.int32, (_SB, 1024), 1)
            j = (((r * _RB + s * _SB) + row) * 1024 + col).astype(jnp.uint32)
            keep = _threefry_bits(j, k0, k1) < THRESH
            e_s = emb_ref[sl, :] * _INV_KEEP
            samp = jnp.where(keep, e_s, np.float32(0.0))
            sbuf[buf, sl, :] = samp
            if m == 0:
                acc_ref[sl, :] = samp
            else:
                acc_ref[sl, :] = acc_ref[sl, :] + samp
        pltpu.async_copy(
            sbuf.at[buf], samp_hbm.at[m, pl.ds(base, _RB), :], sems[buf])

    for buf in (0, 1):
        pltpu.make_async_copy(
            sbuf.at[buf], samp_hbm.at[MC - 2 + buf, pl.ds(base, _RB), :],
            sems[buf]).wait()

    acc = acc_ref[...]
    e_s = emb_ref[...] * _INV_KEEP
    mean = acc / np.float32(MC)
    mean_ref[...] = mean
    std = jnp.sqrt(jnp.maximum(acc * (e_s - mean), 0.0) / np.float32(MC - 1))
    std_ref[...] = std
    ci = lax.broadcasted_iota(jnp.int32, (1024, 16), 0)
    ti = lax.broadcasted_iota(jnp.int32, (1024, 16), 1)
    g = jnp.where(ci // DIM == ti, np.float32(1.0 / DIM), np.float32(0.0))
    tu_ref[...] = lax.dot_general(
        std, g, (((1,), (0,)), ((), ())),
        precision=lax.Precision.HIGHEST,
        preferred_element_type=jnp.float32)


def _tc_moments(emb2d, keys):
    return pl.pallas_call(
        _fused_body,
        grid=(_NBLK,),
        in_specs=[
            pl.BlockSpec((MC, 1, 2), lambda r: (0, 0, 0),
                         memory_space=pltpu.SMEM),
            pl.BlockSpec((_RB, 1024), lambda r: (r, 0)),
        ],
        out_specs=[
            pl.BlockSpec(memory_space=pl.ANY),
            pl.BlockSpec((_RB, 1024), lambda r: (r, 0)),
            pl.BlockSpec((_RB, 1024), lambda r: (r, 0)),
            pl.BlockSpec((_RB, 16), lambda r: (r, 0)),
        ],
        out_shape=[
            jax.ShapeDtypeStruct((MC, N_ROW, 1024), jnp.float32),
            jax.ShapeDtypeStruct((N_ROW, 1024), jnp.float32),
            jax.ShapeDtypeStruct((N_ROW, 1024), jnp.float32),
            jax.ShapeDtypeStruct((N_ROW, 16), jnp.float32),
        ],
        scratch_shapes=[
            pltpu.VMEM((2, _RB, 1024), jnp.float32),
            pltpu.VMEM((_RB, 1024), jnp.float32),
            pltpu.SemaphoreType.DMA,
            pltpu.SemaphoreType.DMA,
        ],
        compiler_params=pltpu.CompilerParams(
            dimension_semantics=("arbitrary",)),
    )(keys, emb2d)


def kernel(input_ids, table):
    ids2d = input_ids.reshape(_NW, _NCHUNK, _CH).astype(jnp.int32)
    keys = jnp.asarray(SAMPLE_KEYS).reshape(MC, 1, 2)
    emb = _sc_gather(ids2d, table)
    emb2d = emb.reshape(N_ROW, 1024)
    samples, mean, std, tu = _tc_moments(emb2d, keys)
    return (mean.reshape(B, S, DIM),
            std.reshape(B, S, DIM),
            tu.reshape(B, S),
            samples.reshape(MC, B, S, DIM))


# speedup vs baseline: 1.0356x; 1.0356x over previous
import functools

import jax
import jax.numpy as jnp
import numpy as np
from jax import lax
from jax.experimental import pallas as pl
from jax.experimental.pallas import tpu as pltpu
from jax.experimental.pallas import tpu_sc as plsc

VOCAB = 1000000
DIM = 64
B, S = 1024, 200
N_TOK = B * S
N_ROW = N_TOK * DIM // 1024
THRESH = np.uint32(5872026 * 512)

SAMPLE_KEYS = np.array([
    (0x6D3E048F, 0x1022172D), (0x03D7B32D, 0xADD083F4),
    (0x92FB20EA, 0x0F38D913), (0xBAD56946, 0x354BA891),
    (0xB013AEE3, 0xC34EDDF6), (0xA4D91A96, 0x3122544E),
    (0xA506C508, 0xB6207291), (0x97D0552F, 0x51BF719F),
    (0x3C999167, 0x8E776FEA), (0x4349448B, 0x92D8BF3B),
], dtype=np.uint32)

_NC, _NS = 2, 16
_NW = _NC * _NS
_RPW = N_TOK // _NW
_CH = 128
_NCHUNK = _RPW // _CH


def _sc_gather(ids2d, table):
    mesh = plsc.VectorSubcoreMesh(core_axis_name="c", subcore_axis_name="s")

    @functools.partial(
        pl.kernel,
        mesh=mesh,
        out_type=jax.ShapeDtypeStruct((N_TOK, DIM), jnp.float32),
        scratch_types=[
            pltpu.VMEM((_NCHUNK, _CH), jnp.int32),
            pltpu.VMEM((2, _CH, DIM), jnp.float32),
            pltpu.SemaphoreType.DMA,
            pltpu.SemaphoreType.DMA,
        ],
        compiler_params=pltpu.CompilerParams(use_tc_tiling_on_sc=False),
    )
    def sc_kernel(ids_hbm, table_hbm, out_hbm, idx_v, rows_v, sem0, sem1):
        w = lax.axis_index("s") * _NC + lax.axis_index("c")
        pltpu.sync_copy(ids_hbm.at[w], idx_v)
        out_base = w * _RPW

        def start(k, buf, sem):
            pltpu.async_copy(table_hbm.at[idx_v.at[k]], rows_v.at[buf], sem)

        def drain(k, buf, sem):
            pltpu.make_async_copy(
                table_hbm.at[idx_v.at[0]], rows_v.at[buf], sem).wait()
            off = pl.multiple_of(out_base + k * _CH, _CH)
            pltpu.sync_copy(rows_v.at[buf], out_hbm.at[pl.ds(off, _CH)])

        start(0, 0, sem0)
        start(1, 1, sem1)

        def body(t, _):
            k = 2 * t
            drain(k, 0, sem0)

            @pl.when(k + 2 < _NCHUNK)
            def _():
                start(k + 2, 0, sem0)

            drain(k + 1, 1, sem1)

            @pl.when(k + 3 < _NCHUNK)
            def _():
                start(k + 3, 1, sem1)
            return 0

        lax.fori_loop(0, _NCHUNK // 2, body, 0)

    return sc_kernel(ids2d, table)


_RB = 128
_NBLK = N_ROW // _RB
_SB = 8
MC = 10
_INV_KEEP = np.float32(1.0 / 0.7)


def _threefry_bits(j, k0, k1):
    ks2 = k0 ^ k1 ^ np.uint32(0x1BD11BDA)
    ks = (k0, k1, ks2)
    rot = ((13, 15, 26, 6), (17, 29, 16, 24))
    x0 = jnp.zeros_like(j) + k0
    x1 = j + k1
    for i in range(5):
        for r in rot[i % 2]:
            x0 = x0 + x1
            x1 = ((x1 << r) | (x1 >> (32 - r))) ^ x0
        x0 = x0 + ks[(i + 1) % 3]
        x1 = x1 + (ks[(i + 2) % 3] + np.uint32(i + 1))
    return x0 ^ x1


def _fused_body(keys_ref, emb_ref, samp_ref, mean_ref, std_ref, tu_ref):
    r = pl.program_id(0)
    for s in range(_RB // _SB):
        sl = pl.ds(s * _SB, _SB)
        row = lax.broadcasted_iota(jnp.int32, (_SB, 1024), 0)
        col = lax.broadcasted_iota(jnp.int32, (_SB, 1024), 1)
        j = (((r * _RB + s * _SB) + row) * 1024 + col).astype(jnp.uint32)
        e_s = emb_ref[sl, :] * _INV_KEEP
        acc = jnp.zeros((_SB, 1024), jnp.float32)
        for m in range(MC):
            k0 = keys_ref[m, 0, 0]
            k1 = keys_ref[m, 0, 1]
            keep = _threefry_bits(j, k0, k1) < THRESH
            samp = jnp.where(keep, e_s, np.float32(0.0))
            samp_ref[m, sl, :] = samp
            acc = acc + samp
        mean = acc * np.float32(1.0 / MC)
        mean_ref[sl, :] = mean
        std_ref[sl, :] = jnp.sqrt(
            jnp.maximum(acc * (e_s - mean), 0.0) * np.float32(1.0 / (MC - 1)))

    std_all = std_ref[...]
    ci = lax.broadcasted_iota(jnp.int32, (1024, 16), 0)
    ti = lax.broadcasted_iota(jnp.int32, (1024, 16), 1)
    g = jnp.where(ci // DIM == ti, np.float32(1.0 / DIM), np.float32(0.0))
    tu_ref[...] = lax.dot_general(
        std_all, g, (((1,), (0,)), ((), ())),
        preferred_element_type=jnp.float32)


def _tc_moments(emb2d, keys):
    return pl.pallas_call(
        _fused_body,
        grid=(_NBLK,),
        in_specs=[
            pl.BlockSpec((MC, 1, 2), lambda r: (0, 0, 0),
                         memory_space=pltpu.SMEM),
            pl.BlockSpec((_RB, 1024), lambda r: (r, 0)),
        ],
        out_specs=[
            pl.BlockSpec((MC, _RB, 1024), lambda r: (0, r, 0)),
            pl.BlockSpec((_RB, 1024), lambda r: (r, 0)),
            pl.BlockSpec((_RB, 1024), lambda r: (r, 0)),
            pl.BlockSpec((_RB, 16), lambda r: (r, 0)),
        ],
        out_shape=[
            jax.ShapeDtypeStruct((MC, N_ROW, 1024), jnp.float32),
            jax.ShapeDtypeStruct((N_ROW, 1024), jnp.float32),
            jax.ShapeDtypeStruct((N_ROW, 1024), jnp.float32),
            jax.ShapeDtypeStruct((N_ROW, 16), jnp.float32),
        ],
        compiler_params=pltpu.CompilerParams(
            dimension_semantics=("arbitrary",)),
    )(keys, emb2d)


def kernel(input_ids, table):
    ids2d = input_ids.reshape(_NW, _NCHUNK, _CH).astype(jnp.int32)
    keys = jnp.asarray(SAMPLE_KEYS).reshape(MC, 1, 2)
    emb = _sc_gather(ids2d, table)
    emb2d = emb.reshape(N_ROW, 1024)
    samples, mean, std, tu = _tc_moments(emb2d, keys)
    return (mean.reshape(B, S, DIM),
            std.reshape(B, S, DIM),
            tu.reshape(B, S),
            samples.reshape(MC, B, S, DIM))
